# Initial kernel scaffold; baseline (speedup 1.0000x reference)
import jax
import jax.numpy as jnp
from jax.experimental import pallas as pl


def kernel(x, edge_index, W1, b1, W2, b2):
    raise NotImplementedError("write your pallas kernel here")


# speedup vs baseline: 21.4448x; 21.4448x over previous
import functools

import jax
import jax.numpy as jnp
from jax import lax
from jax.experimental import pallas as pl
from jax.experimental.pallas import tpu as pltpu
from jax.experimental.pallas import tpu_sc as plsc

N = 10000
E = 320000
F_IN = 128
HID = 64
C = 16
K = 10
ALPHA = 0.1

NC = 2
NS = 16
NW = NC * NS
EPT = E // NW
CH = 125
NCHUNK = EPT // CH
NP = 10240
RPT = NP // NS
NB = 16
BR = NP // NB


def _mlp_body(x_ref, w1_ref, b1_ref, w2_ref, b2_ref, o_ref):
    h = jnp.dot(x_ref[...], w1_ref[...], preferred_element_type=jnp.float32)
    h = jnp.maximum(h + b1_ref[...], 0.0)
    o_ref[...] = (
        jnp.dot(h, w2_ref[...], preferred_element_type=jnp.float32) + b2_ref[...]
    )


_mlp = pl.pallas_call(
    _mlp_body,
    grid=(10,),
    in_specs=[
        pl.BlockSpec((N // 10, F_IN), lambda i: (i, 0)),
        pl.BlockSpec((F_IN, HID), lambda i: (0, 0)),
        pl.BlockSpec((1, HID), lambda i: (0, 0)),
        pl.BlockSpec((HID, C), lambda i: (0, 0)),
        pl.BlockSpec((1, C), lambda i: (0, 0)),
    ],
    out_specs=pl.BlockSpec((N // 10, C), lambda i: (i, 0)),
    out_shape=jax.ShapeDtypeStruct((N, C), jnp.float32),
)


def _prep_body(agg_ref, h_ref, p_ref, w_ref, q_ref, sinv_ref):
    deg = agg_ref[0] + agg_ref[1] - 1.0
    s = lax.rsqrt(deg)
    p0 = s * h_ref[...]
    p_ref[...] = p0
    w_ref[...] = (1.0 - ALPHA) * s * s
    q_ref[...] = ALPHA * p0
    sinv_ref[...] = jnp.sqrt(deg)


_prep = pl.pallas_call(
    _prep_body,
    grid=(NB,),
    in_specs=[
        pl.BlockSpec((NC, BR, C), lambda i: (0, i, 0)),
        pl.BlockSpec((BR, C), lambda i: (i, 0)),
    ],
    out_specs=[pl.BlockSpec((BR, C), lambda i: (i, 0))] * 4,
    out_shape=[jax.ShapeDtypeStruct((NP, C), jnp.float32)] * 4,
)


def _combine_body(agg_ref, p_ref, w_ref, q_ref, o_ref):
    o_ref[...] = (
        w_ref[...] * (agg_ref[0] + agg_ref[1] - p_ref[...]) + q_ref[...]
    )


_combine = pl.pallas_call(
    _combine_body,
    grid=(NB,),
    in_specs=[
        pl.BlockSpec((NC, BR, C), lambda i: (0, i, 0)),
        pl.BlockSpec((BR, C), lambda i: (i, 0)),
        pl.BlockSpec((BR, C), lambda i: (i, 0)),
        pl.BlockSpec((BR, C), lambda i: (i, 0)),
    ],
    out_specs=pl.BlockSpec((BR, C), lambda i: (i, 0)),
    out_shape=jax.ShapeDtypeStruct((NP, C), jnp.float32),
)


def _final_body(p_ref, sinv_ref, o_ref):
    z = p_ref[...] * sinv_ref[...]
    m = jnp.max(z, axis=1, keepdims=True)
    e = jnp.exp(z - m)
    o_ref[...] = (z - m) - jnp.log(jnp.sum(e, axis=1, keepdims=True))


_final = pl.pallas_call(
    _final_body,
    grid=(NB,),
    in_specs=[
        pl.BlockSpec((BR, C), lambda i: (i, 0)),
        pl.BlockSpec((BR, C), lambda i: (i, 0)),
    ],
    out_specs=pl.BlockSpec((BR, C), lambda i: (i, 0)),
    out_shape=jax.ShapeDtypeStruct((NP, C), jnp.float32),
)


@functools.partial(
    pl.kernel,
    out_type=jax.ShapeDtypeStruct((NC, NP, C), jnp.float32),
    mesh=plsc.VectorSubcoreMesh(core_axis_name="c", subcore_axis_name="s"),
    compiler_params=pltpu.CompilerParams(use_tc_tiling_on_sc=False),
    scratch_types=[
        pltpu.VMEM((NCHUNK, CH), jnp.int32),
        pltpu.VMEM((NCHUNK, CH), jnp.int32),
        pltpu.VMEM((CH, C), jnp.float32),
        pltpu.VMEM_SHARED((NP, C), jnp.float32),
    ],
)
def _sc_round(row_hbm, col_hbm, p_hbm, out_hbm, rowi, coli, gbuf, agg):
    c = lax.axis_index("c")
    s = lax.axis_index("s")
    wid = c * NS + s

    pltpu.sync_copy(row_hbm.at[wid], rowi)
    pltpu.sync_copy(col_hbm.at[wid], coli)

    rs = s * RPT
    pltpu.sync_copy(p_hbm.at[pl.ds(rs, RPT)], agg.at[pl.ds(rs, RPT)])
    plsc.subcore_barrier()

    def chunk(j, carry):
        pltpu.sync_copy(p_hbm.at[rowi.at[j]], gbuf)
        pltpu.sync_copy(gbuf, agg.at[coli.at[j]], add=True)
        return carry

    lax.fori_loop(0, NCHUNK, chunk, 0)
    plsc.subcore_barrier()

    pltpu.sync_copy(agg.at[pl.ds(rs, RPT)], out_hbm.at[c, pl.ds(rs, RPT)])


def kernel(x, edge_index, W1, b1, W2, b2):
    row = edge_index[0].reshape(NW, NCHUNK, CH)
    col = edge_index[1].reshape(NW, NCHUNK, CH)

    h = _mlp(x, W1.T, b1.reshape(1, HID), W2.T, b2.reshape(1, C))
    h = jnp.pad(h, ((0, NP - N), (0, 0)))

    ones = jnp.ones((NP, C), jnp.float32)
    cnt_agg = _sc_round(row, col, ones)
    p, w16, q16, sinv = _prep(cnt_agg, h)

    for _ in range(K):
        agg = _sc_round(row, col, p)
        p = _combine(agg, p, w16, q16)

    return _final(p, sinv)[:N]


# speedup vs baseline: 39.7023x; 1.8514x over previous
import functools

import jax
import jax.numpy as jnp
from jax import lax
from jax.experimental import pallas as pl
from jax.experimental.pallas import tpu as pltpu
from jax.experimental.pallas import tpu_sc as plsc

N = 10000
E = 320000
F_IN = 128
HID = 64
C = 16
K = 10
ALPHA = 0.1

NC = 2
NS = 16
NW = NC * NS
EPT = E // NW
CH = 125
NCHUNK = EPT // CH
NG = 8
NGRP = NCHUNK // NG
NP = 10240
RPT = NP // NS
NB = 16
BR = NP // NB


def _mlp_body(x_ref, w1_ref, b1_ref, w2_ref, b2_ref, o_ref):
    h = jnp.dot(x_ref[...], w1_ref[...], preferred_element_type=jnp.float32)
    h = jnp.maximum(h + b1_ref[...], 0.0)
    o_ref[...] = (
        jnp.dot(h, w2_ref[...], preferred_element_type=jnp.float32) + b2_ref[...]
    )


_mlp = pl.pallas_call(
    _mlp_body,
    grid=(10,),
    in_specs=[
        pl.BlockSpec((N // 10, F_IN), lambda i: (i, 0)),
        pl.BlockSpec((F_IN, HID), lambda i: (0, 0)),
        pl.BlockSpec((1, HID), lambda i: (0, 0)),
        pl.BlockSpec((HID, C), lambda i: (0, 0)),
        pl.BlockSpec((1, C), lambda i: (0, 0)),
    ],
    out_specs=pl.BlockSpec((N // 10, C), lambda i: (i, 0)),
    out_shape=jax.ShapeDtypeStruct((N, C), jnp.float32),
)


def _prep_body(agg_ref, h_ref, p_ref, w_ref, q_ref, sinv_ref):
    deg = agg_ref[0] + agg_ref[1] - 1.0
    s = lax.rsqrt(deg)
    p0 = s * h_ref[...]
    p_ref[...] = p0
    w_ref[...] = (1.0 - ALPHA) * s * s
    q_ref[...] = ALPHA * p0
    sinv_ref[...] = jnp.sqrt(deg)


_prep = pl.pallas_call(
    _prep_body,
    grid=(NB,),
    in_specs=[
        pl.BlockSpec((NC, BR, C), lambda i: (0, i, 0)),
        pl.BlockSpec((BR, C), lambda i: (i, 0)),
    ],
    out_specs=[pl.BlockSpec((BR, C), lambda i: (i, 0))] * 4,
    out_shape=[jax.ShapeDtypeStruct((NP, C), jnp.float32)] * 4,
)


def _combine_body(agg_ref, p_ref, w_ref, q_ref, o_ref):
    o_ref[...] = (
        w_ref[...] * (agg_ref[0] + agg_ref[1] - p_ref[...]) + q_ref[...]
    )


_combine = pl.pallas_call(
    _combine_body,
    grid=(NB,),
    in_specs=[
        pl.BlockSpec((NC, BR, C), lambda i: (0, i, 0)),
        pl.BlockSpec((BR, C), lambda i: (i, 0)),
        pl.BlockSpec((BR, C), lambda i: (i, 0)),
        pl.BlockSpec((BR, C), lambda i: (i, 0)),
    ],
    out_specs=pl.BlockSpec((BR, C), lambda i: (i, 0)),
    out_shape=jax.ShapeDtypeStruct((NP, C), jnp.float32),
)


def _final_body(p_ref, sinv_ref, o_ref):
    z = p_ref[...] * sinv_ref[...]
    m = jnp.max(z, axis=1, keepdims=True)
    e = jnp.exp(z - m)
    o_ref[...] = (z - m) - jnp.log(jnp.sum(e, axis=1, keepdims=True))


_final = pl.pallas_call(
    _final_body,
    grid=(NB,),
    in_specs=[
        pl.BlockSpec((BR, C), lambda i: (i, 0)),
        pl.BlockSpec((BR, C), lambda i: (i, 0)),
    ],
    out_specs=pl.BlockSpec((BR, C), lambda i: (i, 0)),
    out_shape=jax.ShapeDtypeStruct((NP, C), jnp.float32),
)


@functools.partial(
    pl.kernel,
    out_type=jax.ShapeDtypeStruct((NC, NP, C), jnp.float32),
    mesh=plsc.VectorSubcoreMesh(core_axis_name="c", subcore_axis_name="s"),
    compiler_params=pltpu.CompilerParams(use_tc_tiling_on_sc=False),
    scratch_types=[
        pltpu.VMEM((NCHUNK, CH), jnp.int32),
        pltpu.VMEM((NCHUNK, CH), jnp.int32),
        pltpu.VMEM((2, NG, CH, C), jnp.float32),
        pltpu.VMEM_SHARED((NP, C), jnp.float32),
        pltpu.SemaphoreType.DMA,
        pltpu.SemaphoreType.DMA,
    ],
)
def _sc_round(row_hbm, col_hbm, p_hbm, out_hbm, rowi, coli, gbuf, agg,
              gsem, ssem):
    c = lax.axis_index("c")
    s = lax.axis_index("s")
    wid = c * NS + s

    pltpu.sync_copy(row_hbm.at[wid], rowi)
    pltpu.sync_copy(col_hbm.at[wid], coli)

    rs = s * RPT
    pltpu.sync_copy(p_hbm.at[pl.ds(rs, RPT)], agg.at[pl.ds(rs, RPT)])
    plsc.subcore_barrier()

    def fire_gathers(g):
        return [
            pltpu.async_copy(p_hbm.at[rowi.at[g * NG + b]],
                             gbuf.at[g % 2, b], gsem)
            for b in range(NG)
        ]

    pend_g = {0: fire_gathers(0)}
    pend_s = {}
    for g in range(NGRP):
        if g + 1 < NGRP:
            if g - 1 in pend_s:
                for d in pend_s.pop(g - 1):
                    d.wait()
            pend_g[g + 1] = fire_gathers(g + 1)
        for d in pend_g.pop(g):
            d.wait()
        pend_s[g] = [
            pltpu.async_copy(gbuf.at[g % 2, b], agg.at[coli.at[g * NG + b]],
                             ssem, add=True)
            for b in range(NG)
        ]
    for g in sorted(pend_s):
        for d in pend_s[g]:
            d.wait()
    plsc.subcore_barrier()

    pltpu.sync_copy(agg.at[pl.ds(rs, RPT)], out_hbm.at[c, pl.ds(rs, RPT)])


def kernel(x, edge_index, W1, b1, W2, b2):
    row = edge_index[0].reshape(NW, NCHUNK, CH)
    col = edge_index[1].reshape(NW, NCHUNK, CH)

    h = _mlp(x, W1.T, b1.reshape(1, HID), W2.T, b2.reshape(1, C))
    h = jnp.pad(h, ((0, NP - N), (0, 0)))

    ones = jnp.ones((NP, C), jnp.float32)
    cnt_agg = _sc_round(row, col, ones)
    p, w16, q16, sinv = _prep(cnt_agg, h)

    for _ in range(K):
        agg = _sc_round(row, col, p)
        p = _combine(agg, p, w16, q16)

    return _final(p, sinv)[:N]


# speedup vs baseline: 41.5498x; 1.0465x over previous
import functools

import jax
import jax.numpy as jnp
from jax import lax
from jax.experimental import pallas as pl
from jax.experimental.pallas import tpu as pltpu
from jax.experimental.pallas import tpu_sc as plsc

N = 10000
E = 320000
F_IN = 128
HID = 64
C = 16
K = 10
ALPHA = 0.1

NC = 2
NS = 16
NW = NC * NS
EPT = E // NW
CH = 500
NCHUNK = EPT // CH
NG = 2
NGRP = NCHUNK // NG
NP = 10240
RPT = NP // NS
NB = 16
BR = NP // NB


def _mlp_body(x_ref, w1_ref, b1_ref, w2_ref, b2_ref, o_ref):
    h = jnp.dot(x_ref[...], w1_ref[...], preferred_element_type=jnp.float32)
    h = jnp.maximum(h + b1_ref[...], 0.0)
    o_ref[...] = (
        jnp.dot(h, w2_ref[...], preferred_element_type=jnp.float32) + b2_ref[...]
    )


_mlp = pl.pallas_call(
    _mlp_body,
    grid=(10,),
    in_specs=[
        pl.BlockSpec((N // 10, F_IN), lambda i: (i, 0)),
        pl.BlockSpec((F_IN, HID), lambda i: (0, 0)),
        pl.BlockSpec((1, HID), lambda i: (0, 0)),
        pl.BlockSpec((HID, C), lambda i: (0, 0)),
        pl.BlockSpec((1, C), lambda i: (0, 0)),
    ],
    out_specs=pl.BlockSpec((N // 10, C), lambda i: (i, 0)),
    out_shape=jax.ShapeDtypeStruct((N, C), jnp.float32),
)


def _prep_body(agg_ref, h_ref, p_ref, w_ref, q_ref, sinv_ref):
    deg = agg_ref[0] + agg_ref[1] - 1.0
    s = lax.rsqrt(deg)
    p0 = s * h_ref[...]
    p_ref[...] = p0
    w_ref[...] = (1.0 - ALPHA) * s * s
    q_ref[...] = ALPHA * p0
    sinv_ref[...] = jnp.sqrt(deg)


_prep = pl.pallas_call(
    _prep_body,
    grid=(NB,),
    in_specs=[
        pl.BlockSpec((NC, BR, C), lambda i: (0, i, 0)),
        pl.BlockSpec((BR, C), lambda i: (i, 0)),
    ],
    out_specs=[pl.BlockSpec((BR, C), lambda i: (i, 0))] * 4,
    out_shape=[jax.ShapeDtypeStruct((NP, C), jnp.float32)] * 4,
)


def _combine_body(agg_ref, p_ref, w_ref, q_ref, o_ref):
    o_ref[...] = (
        w_ref[...] * (agg_ref[0] + agg_ref[1] - p_ref[...]) + q_ref[...]
    )


_combine = pl.pallas_call(
    _combine_body,
    grid=(NB,),
    in_specs=[
        pl.BlockSpec((NC, BR, C), lambda i: (0, i, 0)),
        pl.BlockSpec((BR, C), lambda i: (i, 0)),
        pl.BlockSpec((BR, C), lambda i: (i, 0)),
        pl.BlockSpec((BR, C), lambda i: (i, 0)),
    ],
    out_specs=pl.BlockSpec((BR, C), lambda i: (i, 0)),
    out_shape=jax.ShapeDtypeStruct((NP, C), jnp.float32),
)


def _final_body(p_ref, sinv_ref, o_ref):
    z = p_ref[...] * sinv_ref[...]
    m = jnp.max(z, axis=1, keepdims=True)
    e = jnp.exp(z - m)
    o_ref[...] = (z - m) - jnp.log(jnp.sum(e, axis=1, keepdims=True))


_final = pl.pallas_call(
    _final_body,
    grid=(NB,),
    in_specs=[
        pl.BlockSpec((BR, C), lambda i: (i, 0)),
        pl.BlockSpec((BR, C), lambda i: (i, 0)),
    ],
    out_specs=pl.BlockSpec((BR, C), lambda i: (i, 0)),
    out_shape=jax.ShapeDtypeStruct((NP, C), jnp.float32),
)


@functools.partial(
    pl.kernel,
    out_type=jax.ShapeDtypeStruct((NC, NP, C), jnp.float32),
    mesh=plsc.VectorSubcoreMesh(core_axis_name="c", subcore_axis_name="s"),
    compiler_params=pltpu.CompilerParams(use_tc_tiling_on_sc=False),
    scratch_types=[
        pltpu.VMEM((NCHUNK, CH), jnp.int32),
        pltpu.VMEM((NCHUNK, CH), jnp.int32),
        pltpu.VMEM((2, NG, CH, C), jnp.float32),
        pltpu.VMEM_SHARED((NP, C), jnp.float32),
        pltpu.SemaphoreType.DMA,
        pltpu.SemaphoreType.DMA,
    ],
)
def _sc_round(row_hbm, col_hbm, p_hbm, out_hbm, rowi, coli, gbuf, agg,
              gsem, ssem):
    c = lax.axis_index("c")
    s = lax.axis_index("s")
    wid = c * NS + s

    pltpu.sync_copy(row_hbm.at[wid], rowi)
    pltpu.sync_copy(col_hbm.at[wid], coli)

    rs = s * RPT
    pltpu.sync_copy(p_hbm.at[pl.ds(rs, RPT)], agg.at[pl.ds(rs, RPT)])
    plsc.subcore_barrier()

    def fire_gathers(g):
        return [
            pltpu.async_copy(p_hbm.at[rowi.at[g * NG + b]],
                             gbuf.at[g % 2, b], gsem)
            for b in range(NG)
        ]

    pend_g = {0: fire_gathers(0)}
    pend_s = {}
    for g in range(NGRP):
        if g + 1 < NGRP:
            if g - 1 in pend_s:
                for d in pend_s.pop(g - 1):
                    d.wait()
            pend_g[g + 1] = fire_gathers(g + 1)
        for d in pend_g.pop(g):
            d.wait()
        pend_s[g] = [
            pltpu.async_copy(gbuf.at[g % 2, b], agg.at[coli.at[g * NG + b]],
                             ssem, add=True)
            for b in range(NG)
        ]
    for g in sorted(pend_s):
        for d in pend_s[g]:
            d.wait()
    plsc.subcore_barrier()

    pltpu.sync_copy(agg.at[pl.ds(rs, RPT)], out_hbm.at[c, pl.ds(rs, RPT)])


def kernel(x, edge_index, W1, b1, W2, b2):
    row = edge_index[0].reshape(NW, NCHUNK, CH)
    col = edge_index[1].reshape(NW, NCHUNK, CH)

    h = _mlp(x, W1.T, b1.reshape(1, HID), W2.T, b2.reshape(1, C))
    h = jnp.pad(h, ((0, NP - N), (0, 0)))

    ones = jnp.ones((NP, C), jnp.float32)
    cnt_agg = _sc_round(row, col, ones)
    p, w16, q16, sinv = _prep(cnt_agg, h)

    for _ in range(K):
        agg = _sc_round(row, col, p)
        p = _combine(agg, p, w16, q16)

    return _final(p, sinv)[:N]


# speedup vs baseline: 59.8188x; 1.4397x over previous
import functools

import jax
import jax.numpy as jnp
from jax import lax
from jax.experimental import pallas as pl
from jax.experimental.pallas import tpu as pltpu
from jax.experimental.pallas import tpu_sc as plsc

N = 10000
E = 320000
F_IN = 128
HID = 64
C = 16
K = 10
ALPHA = 0.1

NC = 2
NS = 16
NW = NC * NS
EPT = E // NW
CH = 500
NCHUNK = EPT // CH
NG = 2
NGRP = NCHUNK // NG
NP = 10240
RPT = NP // NS
NB = 16
BR = NP // NB


def _mlp_body(x_ref, w1_ref, b1_ref, w2_ref, b2_ref, o_ref):
    h = jnp.dot(x_ref[...], w1_ref[...], preferred_element_type=jnp.float32)
    h = jnp.maximum(h + b1_ref[...], 0.0)
    o_ref[...] = (
        jnp.dot(h, w2_ref[...], preferred_element_type=jnp.float32) + b2_ref[...]
    )


_mlp = pl.pallas_call(
    _mlp_body,
    grid=(10,),
    in_specs=[
        pl.BlockSpec((N // 10, F_IN), lambda i: (i, 0)),
        pl.BlockSpec((F_IN, HID), lambda i: (0, 0)),
        pl.BlockSpec((1, HID), lambda i: (0, 0)),
        pl.BlockSpec((HID, C), lambda i: (0, 0)),
        pl.BlockSpec((1, C), lambda i: (0, 0)),
    ],
    out_specs=pl.BlockSpec((N // 10, C), lambda i: (i, 0)),
    out_shape=jax.ShapeDtypeStruct((N, C), jnp.float32),
)


def _prep_body(agg_ref, h_ref, p_ref, w_ref, q_ref, sinv_ref):
    deg = agg_ref[0] + agg_ref[1] - 1.0
    s = lax.rsqrt(deg)
    p0 = s * h_ref[...]
    p_ref[...] = p0
    w_ref[...] = (1.0 - ALPHA) * s * s
    q_ref[...] = ALPHA * p0
    sinv_ref[...] = jnp.sqrt(deg)


_prep = pl.pallas_call(
    _prep_body,
    grid=(NB,),
    in_specs=[
        pl.BlockSpec((NC, BR, C), lambda i: (0, i, 0)),
        pl.BlockSpec((BR, C), lambda i: (i, 0)),
    ],
    out_specs=[pl.BlockSpec((BR, C), lambda i: (i, 0))] * 4,
    out_shape=[jax.ShapeDtypeStruct((NP, C), jnp.float32)] * 4,
)


def _combine_body(agg_ref, p_ref, w_ref, q_ref, o_ref):
    o_ref[...] = (
        w_ref[...] * (agg_ref[0] + agg_ref[1] - p_ref[...]) + q_ref[...]
    )


_combine = pl.pallas_call(
    _combine_body,
    grid=(NB,),
    in_specs=[
        pl.BlockSpec((NC, BR, C), lambda i: (0, i, 0)),
        pl.BlockSpec((BR, C), lambda i: (i, 0)),
        pl.BlockSpec((BR, C), lambda i: (i, 0)),
        pl.BlockSpec((BR, C), lambda i: (i, 0)),
    ],
    out_specs=pl.BlockSpec((BR, C), lambda i: (i, 0)),
    out_shape=jax.ShapeDtypeStruct((NP, C), jnp.float32),
)


def _final_body(p_ref, sinv_ref, o_ref):
    z = p_ref[...] * sinv_ref[...]
    m = jnp.max(z, axis=1, keepdims=True)
    e = jnp.exp(z - m)
    o_ref[...] = (z - m) - jnp.log(jnp.sum(e, axis=1, keepdims=True))


_final = pl.pallas_call(
    _final_body,
    grid=(NB,),
    in_specs=[
        pl.BlockSpec((BR, C), lambda i: (i, 0)),
        pl.BlockSpec((BR, C), lambda i: (i, 0)),
    ],
    out_specs=pl.BlockSpec((BR, C), lambda i: (i, 0)),
    out_shape=jax.ShapeDtypeStruct((NP, C), jnp.float32),
)


@functools.partial(
    pl.kernel,
    out_type=jax.ShapeDtypeStruct((NC, NP, C), jnp.float32),
    mesh=plsc.VectorSubcoreMesh(core_axis_name="c", subcore_axis_name="s"),
    compiler_params=pltpu.CompilerParams(use_tc_tiling_on_sc=False),
    scratch_types=[
        pltpu.VMEM((NCHUNK, CH), jnp.int32),
        pltpu.VMEM((NCHUNK, CH), jnp.int32),
        pltpu.VMEM((2, NG, CH, C), jnp.float32),
        pltpu.VMEM_SHARED((NP, C), jnp.float32),
        pltpu.SemaphoreType.DMA,
        pltpu.SemaphoreType.DMA,
    ],
)
def _sc_round(row_hbm, col_hbm, p_hbm, out_hbm, rowi, coli, gbuf, agg,
              gsem, ssem):
    c = lax.axis_index("c")
    s = lax.axis_index("s")
    wid = c * NS + s

    pltpu.sync_copy(row_hbm.at[wid], rowi)
    pltpu.sync_copy(col_hbm.at[wid], coli)

    rs = s * RPT
    pltpu.sync_copy(p_hbm.at[pl.ds(rs, RPT)], agg.at[pl.ds(rs, RPT)])
    plsc.subcore_barrier()

    def fire_gathers(g):
        return [
            pltpu.async_copy(p_hbm.at[rowi.at[g * NG + b]],
                             gbuf.at[g % 2, b], gsem)
            for b in range(NG)
        ]

    pend_g = {0: fire_gathers(0)}
    pend_s = {}
    for g in range(NGRP):
        if g + 1 < NGRP:
            if g - 1 in pend_s:
                for d in pend_s.pop(g - 1):
                    d.wait()
            pend_g[g + 1] = fire_gathers(g + 1)
        for d in pend_g.pop(g):
            d.wait()
        pend_s[g] = [
            pltpu.async_copy(gbuf.at[g % 2, b], agg.at[coli.at[g * NG + b]],
                             ssem, add=True)
            for b in range(NG)
        ]
    for g in sorted(pend_s):
        for d in pend_s[g]:
            d.wait()
    plsc.subcore_barrier()

    pltpu.sync_copy(agg.at[pl.ds(rs, RPT)], out_hbm.at[c, pl.ds(rs, RPT)])


@functools.partial(
    pl.kernel,
    out_type=[
        jax.ShapeDtypeStruct((NP, C), jnp.float32),
        jax.ShapeDtypeStruct((2, NC, NP, C), jnp.float32),
    ],
    mesh=plsc.VectorSubcoreMesh(core_axis_name="c", subcore_axis_name="s"),
    compiler_params=pltpu.CompilerParams(use_tc_tiling_on_sc=False),
    scratch_types=[
        pltpu.VMEM((NCHUNK, CH), jnp.int32),
        pltpu.VMEM((NCHUNK, CH), jnp.int32),
        pltpu.VMEM((2, CH, C), jnp.float32),
        pltpu.VMEM((RPT, C), jnp.float32),
        pltpu.VMEM((RPT, C), jnp.float32),
        pltpu.VMEM((RPT, C), jnp.float32),
        pltpu.VMEM((RPT, C), jnp.float32),
        pltpu.VMEM((RPT, C), jnp.float32),
        pltpu.VMEM_SHARED((NP, C), jnp.float32),
        pltpu.SemaphoreType.DMA,
        pltpu.SemaphoreType.DMA,
        pltpu.SemaphoreType.REGULAR,
    ],
)
def _sc_prop(row_hbm, col_hbm, p0_hbm, w_hbm, q_hbm, pout_hbm, x_hbm,
             rowi, coli, gbuf, pvm, wvm, qvm, avm, xvm, agg,
             gsem, ssem, xsem):
    c = lax.axis_index("c")
    s = lax.axis_index("s")
    wid = c * NS + s
    rs = s * RPT
    stripe = pl.ds(rs, RPT)

    pltpu.sync_copy(row_hbm.at[wid], rowi)
    pltpu.sync_copy(col_hbm.at[wid], coli)
    pltpu.sync_copy(w_hbm.at[stripe], wvm)
    pltpu.sync_copy(q_hbm.at[stripe], qvm)

    for r in range(K):
        src = p0_hbm if r == 0 else pout_hbm

        pltpu.sync_copy(src.at[stripe], pvm)
        pltpu.sync_copy(pvm, agg.at[stripe])
        plsc.subcore_barrier()

        def fire(j):
            return pltpu.async_copy(src.at[rowi.at[j]], gbuf.at[j % 2], gsem)

        pend_g = {0: fire(0)}
        pend_s = {}
        for j in range(NCHUNK):
            if j + 1 < NCHUNK:
                if j - 1 in pend_s:
                    pend_s.pop(j - 1).wait()
                pend_g[j + 1] = fire(j + 1)
            pend_g.pop(j).wait()
            pend_s[j] = pltpu.async_copy(
                gbuf.at[j % 2], agg.at[coli.at[j]], ssem, add=True)
        for j in sorted(pend_s):
            pend_s[j].wait()
        plsc.subcore_barrier()

        pltpu.sync_copy(agg.at[stripe], avm)
        pltpu.sync_copy(avm, x_hbm.at[r % 2, c, stripe])
        pl.semaphore_signal(xsem, 1, core_index=1 - c)
        pl.semaphore_wait(xsem, 1)
        pltpu.sync_copy(x_hbm.at[r % 2, 1 - c, stripe], xvm)

        def dense_row(i, carry):
            pvm[i] = wvm[i] * (avm[i] + xvm[i] - pvm[i]) + qvm[i]
            return carry

        lax.fori_loop(0, RPT, dense_row, 0)
        pltpu.sync_copy(pvm, pout_hbm.at[stripe])
        plsc.subcore_barrier()


def kernel(x, edge_index, W1, b1, W2, b2):
    row = edge_index[0].reshape(NW, NCHUNK, CH)
    col = edge_index[1].reshape(NW, NCHUNK, CH)

    h = _mlp(x, W1.T, b1.reshape(1, HID), W2.T, b2.reshape(1, C))
    h = jnp.pad(h, ((0, NP - N), (0, 0)))

    ones = jnp.ones((NP, C), jnp.float32)
    cnt_agg = _sc_round(row, col, ones)
    p0, w16, q16, sinv = _prep(cnt_agg, h)

    p, _ = _sc_prop(row, col, p0, w16, q16)

    return _final(p, sinv)[:N]


# speedup vs baseline: 66.2772x; 1.1080x over previous
import functools

import jax
import jax.numpy as jnp
from jax import lax
from jax.experimental import pallas as pl
from jax.experimental.pallas import tpu as pltpu
from jax.experimental.pallas import tpu_sc as plsc

N = 10000
E = 320000
F_IN = 128
HID = 64
C = 16
K = 10
ALPHA = 0.1

NC = 2
NS = 16
NW = NC * NS
EPT = E // NW
CH = 1000
NCHUNK = EPT // CH
NG = 1
NGRP = NCHUNK // NG
NP = 10240
RPT = NP // NS
NB = 16
BR = NP // NB


def _mlp_body(x_ref, w1_ref, b1_ref, w2_ref, b2_ref, o_ref):
    h = jnp.dot(x_ref[...], w1_ref[...], preferred_element_type=jnp.float32)
    h = jnp.maximum(h + b1_ref[...], 0.0)
    o_ref[...] = (
        jnp.dot(h, w2_ref[...], preferred_element_type=jnp.float32) + b2_ref[...]
    )


_mlp = pl.pallas_call(
    _mlp_body,
    grid=(10,),
    in_specs=[
        pl.BlockSpec((N // 10, F_IN), lambda i: (i, 0)),
        pl.BlockSpec((F_IN, HID), lambda i: (0, 0)),
        pl.BlockSpec((1, HID), lambda i: (0, 0)),
        pl.BlockSpec((HID, C), lambda i: (0, 0)),
        pl.BlockSpec((1, C), lambda i: (0, 0)),
    ],
    out_specs=pl.BlockSpec((N // 10, C), lambda i: (i, 0)),
    out_shape=jax.ShapeDtypeStruct((N, C), jnp.float32),
)


def _prep_body(agg_ref, h_ref, p_ref, w_ref, q_ref, sinv_ref):
    deg = agg_ref[0] + agg_ref[1] - 1.0
    s = lax.rsqrt(deg)
    p0 = s * h_ref[...]
    p_ref[...] = p0
    w_ref[...] = (1.0 - ALPHA) * s * s
    q_ref[...] = ALPHA * p0
    sinv_ref[...] = jnp.sqrt(deg)


_prep = pl.pallas_call(
    _prep_body,
    grid=(NB,),
    in_specs=[
        pl.BlockSpec((NC, BR, C), lambda i: (0, i, 0)),
        pl.BlockSpec((BR, C), lambda i: (i, 0)),
    ],
    out_specs=[pl.BlockSpec((BR, C), lambda i: (i, 0))] * 4,
    out_shape=[jax.ShapeDtypeStruct((NP, C), jnp.float32)] * 4,
)


def _combine_body(agg_ref, p_ref, w_ref, q_ref, o_ref):
    o_ref[...] = (
        w_ref[...] * (agg_ref[0] + agg_ref[1] - p_ref[...]) + q_ref[...]
    )


_combine = pl.pallas_call(
    _combine_body,
    grid=(NB,),
    in_specs=[
        pl.BlockSpec((NC, BR, C), lambda i: (0, i, 0)),
        pl.BlockSpec((BR, C), lambda i: (i, 0)),
        pl.BlockSpec((BR, C), lambda i: (i, 0)),
        pl.BlockSpec((BR, C), lambda i: (i, 0)),
    ],
    out_specs=pl.BlockSpec((BR, C), lambda i: (i, 0)),
    out_shape=jax.ShapeDtypeStruct((NP, C), jnp.float32),
)


def _final_body(p_ref, sinv_ref, o_ref):
    z = p_ref[...] * sinv_ref[...]
    m = jnp.max(z, axis=1, keepdims=True)
    e = jnp.exp(z - m)
    o_ref[...] = (z - m) - jnp.log(jnp.sum(e, axis=1, keepdims=True))


_final = pl.pallas_call(
    _final_body,
    grid=(NB,),
    in_specs=[
        pl.BlockSpec((BR, C), lambda i: (i, 0)),
        pl.BlockSpec((BR, C), lambda i: (i, 0)),
    ],
    out_specs=pl.BlockSpec((BR, C), lambda i: (i, 0)),
    out_shape=jax.ShapeDtypeStruct((NP, C), jnp.float32),
)


@functools.partial(
    pl.kernel,
    out_type=jax.ShapeDtypeStruct((NC, NP, C), jnp.float32),
    mesh=plsc.VectorSubcoreMesh(core_axis_name="c", subcore_axis_name="s"),
    compiler_params=pltpu.CompilerParams(use_tc_tiling_on_sc=False),
    scratch_types=[
        pltpu.VMEM((NCHUNK, CH), jnp.int32),
        pltpu.VMEM((NCHUNK, CH), jnp.int32),
        pltpu.VMEM((2, NG, CH, C), jnp.float32),
        pltpu.VMEM_SHARED((NP, C), jnp.float32),
        pltpu.SemaphoreType.DMA,
        pltpu.SemaphoreType.DMA,
    ],
)
def _sc_round(row_hbm, col_hbm, p_hbm, out_hbm, rowi, coli, gbuf, agg,
              gsem, ssem):
    c = lax.axis_index("c")
    s = lax.axis_index("s")
    wid = c * NS + s

    pltpu.sync_copy(row_hbm.at[wid], rowi)
    pltpu.sync_copy(col_hbm.at[wid], coli)

    rs = s * RPT
    pltpu.sync_copy(p_hbm.at[pl.ds(rs, RPT)], agg.at[pl.ds(rs, RPT)])
    plsc.subcore_barrier()

    def fire_gathers(g):
        return [
            pltpu.async_copy(p_hbm.at[rowi.at[g * NG + b]],
                             gbuf.at[g % 2, b], gsem)
            for b in range(NG)
        ]

    pend_g = {0: fire_gathers(0)}
    pend_s = {}
    for g in range(NGRP):
        if g + 1 < NGRP:
            if g - 1 in pend_s:
                for d in pend_s.pop(g - 1):
                    d.wait()
            pend_g[g + 1] = fire_gathers(g + 1)
        for d in pend_g.pop(g):
            d.wait()
        pend_s[g] = [
            pltpu.async_copy(gbuf.at[g % 2, b], agg.at[coli.at[g * NG + b]],
                             ssem, add=True)
            for b in range(NG)
        ]
    for g in sorted(pend_s):
        for d in pend_s[g]:
            d.wait()
    plsc.subcore_barrier()

    pltpu.sync_copy(agg.at[pl.ds(rs, RPT)], out_hbm.at[c, pl.ds(rs, RPT)])


@functools.partial(
    pl.kernel,
    out_type=[
        jax.ShapeDtypeStruct((NP, C), jnp.float32),
        jax.ShapeDtypeStruct((2, NC, NP, C), jnp.float32),
    ],
    mesh=plsc.VectorSubcoreMesh(core_axis_name="c", subcore_axis_name="s"),
    compiler_params=pltpu.CompilerParams(use_tc_tiling_on_sc=False),
    scratch_types=[
        pltpu.VMEM((NCHUNK, CH), jnp.int32),
        pltpu.VMEM((NCHUNK, CH), jnp.int32),
        pltpu.VMEM((2, CH, C), jnp.float32),
        pltpu.VMEM((RPT, C), jnp.float32),
        pltpu.VMEM((RPT, C), jnp.float32),
        pltpu.VMEM((RPT, C), jnp.float32),
        pltpu.VMEM((RPT, C), jnp.float32),
        pltpu.VMEM((RPT, C), jnp.float32),
        pltpu.VMEM_SHARED((NP, C), jnp.float32),
        pltpu.SemaphoreType.DMA,
        pltpu.SemaphoreType.DMA,
        pltpu.SemaphoreType.REGULAR,
    ],
)
def _sc_prop(row_hbm, col_hbm, p0_hbm, w_hbm, q_hbm, pout_hbm, x_hbm,
             rowi, coli, gbuf, pvm, wvm, qvm, avm, xvm, agg,
             gsem, ssem, xsem):
    c = lax.axis_index("c")
    s = lax.axis_index("s")
    wid = c * NS + s
    rs = s * RPT
    stripe = pl.ds(rs, RPT)

    pltpu.sync_copy(row_hbm.at[wid], rowi)
    pltpu.sync_copy(col_hbm.at[wid], coli)
    pltpu.sync_copy(w_hbm.at[stripe], wvm)
    pltpu.sync_copy(q_hbm.at[stripe], qvm)

    for r in range(K):
        src = p0_hbm if r == 0 else pout_hbm

        pltpu.sync_copy(src.at[stripe], pvm)
        pltpu.sync_copy(pvm, agg.at[stripe])
        plsc.subcore_barrier()

        def fire(j):
            return pltpu.async_copy(src.at[rowi.at[j]], gbuf.at[j % 2], gsem)

        pend_g = {0: fire(0)}
        pend_s = {}
        for j in range(NCHUNK):
            if j + 1 < NCHUNK:
                if j - 1 in pend_s:
                    pend_s.pop(j - 1).wait()
                pend_g[j + 1] = fire(j + 1)
            pend_g.pop(j).wait()
            pend_s[j] = pltpu.async_copy(
                gbuf.at[j % 2], agg.at[coli.at[j]], ssem, add=True)
        for j in sorted(pend_s):
            pend_s[j].wait()
        plsc.subcore_barrier()

        pltpu.sync_copy(agg.at[stripe], avm)
        pltpu.sync_copy(avm, x_hbm.at[r % 2, c, stripe])
        pl.semaphore_signal(xsem, 1, core_index=1 - c)
        pl.semaphore_wait(xsem, 1)
        pltpu.sync_copy(x_hbm.at[r % 2, 1 - c, stripe], xvm)

        def dense_row(i, carry):
            pvm[i] = wvm[i] * (avm[i] + xvm[i] - pvm[i]) + qvm[i]
            return carry

        lax.fori_loop(0, RPT, dense_row, 0)
        pltpu.sync_copy(pvm, pout_hbm.at[stripe])
        plsc.subcore_barrier()


def kernel(x, edge_index, W1, b1, W2, b2):
    row = edge_index[0].reshape(NW, NCHUNK, CH)
    col = edge_index[1].reshape(NW, NCHUNK, CH)

    h = _mlp(x, W1.T, b1.reshape(1, HID), W2.T, b2.reshape(1, C))
    h = jnp.pad(h, ((0, NP - N), (0, 0)))

    ones = jnp.ones((NP, C), jnp.float32)
    cnt_agg = _sc_round(row, col, ones)
    p0, w16, q16, sinv = _prep(cnt_agg, h)

    p, _ = _sc_prop(row, col, p0, w16, q16)

    return _final(p, sinv)[:N]


# speedup vs baseline: 70.3317x; 1.0612x over previous
import functools

import jax
import jax.numpy as jnp
from jax import lax
from jax.experimental import pallas as pl
from jax.experimental.pallas import tpu as pltpu
from jax.experimental.pallas import tpu_sc as plsc

N = 10000
E = 320000
F_IN = 128
HID = 64
C = 16
K = 10
ALPHA = 0.1

NC = 2
NS = 16
NW = NC * NS
EPT = E // NW
CH = 1000
NCHUNK = EPT // CH
NG = 1
NGRP = NCHUNK // NG
NP = 10240
RPT = NP // NS
NB = 16
BR = NP // NB


def _mlp_body(x_ref, w1_ref, b1_ref, w2_ref, b2_ref, o_ref):
    h = jnp.dot(x_ref[...], w1_ref[...], preferred_element_type=jnp.float32)
    h = jnp.maximum(h + b1_ref[...], 0.0)
    o_ref[...] = (
        jnp.dot(h, w2_ref[...], preferred_element_type=jnp.float32) + b2_ref[...]
    )


_mlp = pl.pallas_call(
    _mlp_body,
    grid=(10,),
    in_specs=[
        pl.BlockSpec((N // 10, F_IN), lambda i: (i, 0)),
        pl.BlockSpec((F_IN, HID), lambda i: (0, 0)),
        pl.BlockSpec((1, HID), lambda i: (0, 0)),
        pl.BlockSpec((HID, C), lambda i: (0, 0)),
        pl.BlockSpec((1, C), lambda i: (0, 0)),
    ],
    out_specs=pl.BlockSpec((N // 10, C), lambda i: (i, 0)),
    out_shape=jax.ShapeDtypeStruct((N, C), jnp.float32),
)


def _prep_body(agg_ref, h_ref, p_ref, w_ref, q_ref, sinv_ref):
    deg = agg_ref[0] + agg_ref[1] - 1.0
    s = lax.rsqrt(deg)
    p0 = s * h_ref[...]
    p_ref[...] = p0
    w_ref[...] = (1.0 - ALPHA) * s * s
    q_ref[...] = ALPHA * p0
    sinv_ref[...] = jnp.sqrt(deg)


_prep = pl.pallas_call(
    _prep_body,
    grid=(NB,),
    in_specs=[
        pl.BlockSpec((NC, BR, C), lambda i: (0, i, 0)),
        pl.BlockSpec((BR, C), lambda i: (i, 0)),
    ],
    out_specs=[pl.BlockSpec((BR, C), lambda i: (i, 0))] * 4,
    out_shape=[jax.ShapeDtypeStruct((NP, C), jnp.float32)] * 4,
)


def _combine_body(agg_ref, p_ref, w_ref, q_ref, o_ref):
    o_ref[...] = (
        w_ref[...] * (agg_ref[0] + agg_ref[1] - p_ref[...]) + q_ref[...]
    )


_combine = pl.pallas_call(
    _combine_body,
    grid=(NB,),
    in_specs=[
        pl.BlockSpec((NC, BR, C), lambda i: (0, i, 0)),
        pl.BlockSpec((BR, C), lambda i: (i, 0)),
        pl.BlockSpec((BR, C), lambda i: (i, 0)),
        pl.BlockSpec((BR, C), lambda i: (i, 0)),
    ],
    out_specs=pl.BlockSpec((BR, C), lambda i: (i, 0)),
    out_shape=jax.ShapeDtypeStruct((NP, C), jnp.float32),
)


def _final_body(p_ref, sinv_ref, o_ref):
    z = p_ref[...] * sinv_ref[...]
    m = jnp.max(z, axis=1, keepdims=True)
    e = jnp.exp(z - m)
    o_ref[...] = (z - m) - jnp.log(jnp.sum(e, axis=1, keepdims=True))


_final = pl.pallas_call(
    _final_body,
    grid=(NB,),
    in_specs=[
        pl.BlockSpec((BR, C), lambda i: (i, 0)),
        pl.BlockSpec((BR, C), lambda i: (i, 0)),
    ],
    out_specs=pl.BlockSpec((BR, C), lambda i: (i, 0)),
    out_shape=jax.ShapeDtypeStruct((NP, C), jnp.float32),
)


@functools.partial(
    pl.kernel,
    out_type=jax.ShapeDtypeStruct((NC, NP, C), jnp.float32),
    mesh=plsc.VectorSubcoreMesh(core_axis_name="c", subcore_axis_name="s"),
    compiler_params=pltpu.CompilerParams(use_tc_tiling_on_sc=False),
    scratch_types=[
        pltpu.VMEM((NCHUNK, CH), jnp.int32),
        pltpu.VMEM((NCHUNK, CH), jnp.int32),
        pltpu.VMEM((2, NG, CH, C), jnp.float32),
        pltpu.VMEM_SHARED((NP, C), jnp.float32),
        pltpu.SemaphoreType.DMA,
        pltpu.SemaphoreType.DMA,
    ],
)
def _sc_round(row_hbm, col_hbm, p_hbm, out_hbm, rowi, coli, gbuf, agg,
              gsem, ssem):
    c = lax.axis_index("c")
    s = lax.axis_index("s")
    wid = c * NS + s

    pltpu.sync_copy(row_hbm.at[wid], rowi)
    pltpu.sync_copy(col_hbm.at[wid], coli)

    rs = s * RPT
    pltpu.sync_copy(p_hbm.at[pl.ds(rs, RPT)], agg.at[pl.ds(rs, RPT)])
    plsc.subcore_barrier()

    def fire_gathers(g):
        return [
            pltpu.async_copy(p_hbm.at[rowi.at[g * NG + b]],
                             gbuf.at[g % 2, b], gsem)
            for b in range(NG)
        ]

    pend_g = {0: fire_gathers(0)}
    pend_s = {}
    for g in range(NGRP):
        if g + 1 < NGRP:
            if g - 1 in pend_s:
                for d in pend_s.pop(g - 1):
                    d.wait()
            pend_g[g + 1] = fire_gathers(g + 1)
        for d in pend_g.pop(g):
            d.wait()
        pend_s[g] = [
            pltpu.async_copy(gbuf.at[g % 2, b], agg.at[coli.at[g * NG + b]],
                             ssem, add=True)
            for b in range(NG)
        ]
    for g in sorted(pend_s):
        for d in pend_s[g]:
            d.wait()
    plsc.subcore_barrier()

    pltpu.sync_copy(agg.at[pl.ds(rs, RPT)], out_hbm.at[c, pl.ds(rs, RPT)])


@functools.partial(
    pl.kernel,
    out_type=[
        jax.ShapeDtypeStruct((NP, C), jnp.float32),
        jax.ShapeDtypeStruct((2, NC, NP, C), jnp.float32),
    ],
    mesh=plsc.VectorSubcoreMesh(core_axis_name="c", subcore_axis_name="s"),
    compiler_params=pltpu.CompilerParams(use_tc_tiling_on_sc=False),
    scratch_types=[
        pltpu.VMEM((NCHUNK, CH), jnp.int32),
        pltpu.VMEM((NCHUNK, CH), jnp.int32),
        pltpu.VMEM((3, CH, C), jnp.float32),
        pltpu.VMEM((RPT, C), jnp.float32),
        pltpu.VMEM((RPT, C), jnp.float32),
        pltpu.VMEM((RPT, C), jnp.float32),
        pltpu.VMEM((RPT, C), jnp.float32),
        pltpu.VMEM((RPT, C), jnp.float32),
        pltpu.VMEM_SHARED((NP, C), jnp.float32),
        pltpu.SemaphoreType.DMA,
        pltpu.SemaphoreType.DMA,
        pltpu.SemaphoreType.REGULAR,
    ],
)
def _sc_prop(row_hbm, col_hbm, p0_hbm, w_hbm, q_hbm, pout_hbm, x_hbm,
             rowi, coli, gbuf, pvm, wvm, qvm, avm, xvm, agg,
             gsem, ssem, xsem):
    c = lax.axis_index("c")
    s = lax.axis_index("s")
    wid = c * NS + s
    rs = s * RPT
    stripe = pl.ds(rs, RPT)

    pltpu.sync_copy(row_hbm.at[wid], rowi)
    pltpu.sync_copy(col_hbm.at[wid], coli)
    pltpu.sync_copy(w_hbm.at[stripe], wvm)
    pltpu.sync_copy(q_hbm.at[stripe], qvm)

    for r in range(K):
        src = p0_hbm if r == 0 else pout_hbm

        pltpu.sync_copy(src.at[stripe], pvm)
        pltpu.sync_copy(pvm, agg.at[stripe])
        plsc.subcore_barrier()

        def fire(j):
            return pltpu.async_copy(src.at[rowi.at[j]], gbuf.at[j % 3], gsem)

        pend_g = {0: fire(0), 1: fire(1)}
        pend_s = {}
        for j in range(NCHUNK):
            if j + 2 < NCHUNK:
                if j - 1 in pend_s:
                    pend_s.pop(j - 1).wait()
                pend_g[j + 2] = fire(j + 2)
            pend_g.pop(j).wait()
            pend_s[j] = pltpu.async_copy(
                gbuf.at[j % 3], agg.at[coli.at[j]], ssem, add=True)
        for j in sorted(pend_s):
            pend_s[j].wait()
        plsc.subcore_barrier()

        pltpu.sync_copy(agg.at[stripe], avm)
        pltpu.sync_copy(avm, x_hbm.at[r % 2, c, stripe])
        pl.semaphore_signal(xsem, 1, core_index=1 - c)
        pl.semaphore_wait(xsem, 1)
        pltpu.sync_copy(x_hbm.at[r % 2, 1 - c, stripe], xvm)

        def dense_row(i, carry):
            pvm[i] = wvm[i] * (avm[i] + xvm[i] - pvm[i]) + qvm[i]
            return carry

        lax.fori_loop(0, RPT, dense_row, 0)
        pltpu.sync_copy(pvm, pout_hbm.at[stripe])
        plsc.subcore_barrier()


def kernel(x, edge_index, W1, b1, W2, b2):
    row = edge_index[0].reshape(NW, NCHUNK, CH)
    col = edge_index[1].reshape(NW, NCHUNK, CH)

    h = _mlp(x, W1.T, b1.reshape(1, HID), W2.T, b2.reshape(1, C))
    h = jnp.pad(h, ((0, NP - N), (0, 0)))

    ones = jnp.ones((NP, C), jnp.float32)
    cnt_agg = _sc_round(row, col, ones)
    p0, w16, q16, sinv = _prep(cnt_agg, h)

    p, _ = _sc_prop(row, col, p0, w16, q16)

    return _final(p, sinv)[:N]


# speedup vs baseline: 74.1749x; 1.0546x over previous
import functools

import jax
import jax.numpy as jnp
from jax import lax
from jax.experimental import pallas as pl
from jax.experimental.pallas import tpu as pltpu
from jax.experimental.pallas import tpu_sc as plsc

N = 10000
E = 320000
F_IN = 128
HID = 64
C = 16
K = 10
ALPHA = 0.1

NC = 2
NS = 16
NW = NC * NS
EPT = E // NW
CH = 1000
NCHUNK = EPT // CH
NP = 10240
RPT = NP // NS
NB = 16
BR = NP // NB


def _mlp_body(x_ref, w1_ref, b1_ref, w2_ref, b2_ref, o_ref):
    h = jnp.dot(x_ref[...], w1_ref[...], preferred_element_type=jnp.float32)
    h = jnp.maximum(h + b1_ref[...], 0.0)
    o_ref[...] = (
        jnp.dot(h, w2_ref[...], preferred_element_type=jnp.float32) + b2_ref[...]
    )


_mlp = pl.pallas_call(
    _mlp_body,
    grid=(10,),
    in_specs=[
        pl.BlockSpec((N // 10, F_IN), lambda i: (i, 0)),
        pl.BlockSpec((F_IN, HID), lambda i: (0, 0)),
        pl.BlockSpec((1, HID), lambda i: (0, 0)),
        pl.BlockSpec((HID, C), lambda i: (0, 0)),
        pl.BlockSpec((1, C), lambda i: (0, 0)),
    ],
    out_specs=pl.BlockSpec((N // 10, C), lambda i: (i, 0)),
    out_shape=jax.ShapeDtypeStruct((N, C), jnp.float32),
)


def _final_body(p_ref, sinv_ref, o_ref):
    z = p_ref[...] * sinv_ref[...]
    m = jnp.max(z, axis=1, keepdims=True)
    e = jnp.exp(z - m)
    o_ref[...] = (z - m) - jnp.log(jnp.sum(e, axis=1, keepdims=True))


_final = pl.pallas_call(
    _final_body,
    grid=(NB,),
    in_specs=[
        pl.BlockSpec((BR, C), lambda i: (i, 0)),
        pl.BlockSpec((BR, C), lambda i: (i, 0)),
    ],
    out_specs=pl.BlockSpec((BR, C), lambda i: (i, 0)),
    out_shape=jax.ShapeDtypeStruct((NP, C), jnp.float32),
)


@functools.partial(
    pl.kernel,
    out_type=[
        jax.ShapeDtypeStruct((NP, C), jnp.float32),
        jax.ShapeDtypeStruct((NP, C), jnp.float32),
        jax.ShapeDtypeStruct((2, NC, NP, C), jnp.float32),
    ],
    mesh=plsc.VectorSubcoreMesh(core_axis_name="c", subcore_axis_name="s"),
    compiler_params=pltpu.CompilerParams(use_tc_tiling_on_sc=False),
    scratch_types=[
        pltpu.VMEM((NCHUNK, CH), jnp.int32),
        pltpu.VMEM((NCHUNK, CH), jnp.int32),
        pltpu.VMEM((3, CH, C), jnp.float32),
        pltpu.VMEM((RPT, C), jnp.float32),
        pltpu.VMEM((RPT, C), jnp.float32),
        pltpu.VMEM((RPT, C), jnp.float32),
        pltpu.VMEM((RPT, C), jnp.float32),
        pltpu.VMEM((RPT, C), jnp.float32),
        pltpu.VMEM_SHARED((NP, C), jnp.float32),
        pltpu.SemaphoreType.DMA,
        pltpu.SemaphoreType.DMA,
        pltpu.SemaphoreType.REGULAR,
    ],
)
def _sc_prop(row_hbm, col_hbm, h_hbm, pout_hbm, sinv_hbm, x_hbm,
             rowi, coli, gbuf, pvm, wvm, qvm, avm, xvm, agg,
             gsem, ssem, xsem):
    c = lax.axis_index("c")
    s = lax.axis_index("s")
    wid = c * NS + s
    rs = s * RPT
    stripe = pl.ds(rs, RPT)

    pltpu.sync_copy(row_hbm.at[wid], rowi)
    pltpu.sync_copy(col_hbm.at[wid], coli)

    def exchange(phase):
        pltpu.sync_copy(agg.at[stripe], avm)
        pltpu.sync_copy(avm, x_hbm.at[phase % 2, c, stripe])
        pl.semaphore_signal(xsem, 1, core_index=1 - c)
        pl.semaphore_wait(xsem, 1)
        pltpu.sync_copy(x_hbm.at[phase % 2, 1 - c, stripe], xvm)

    ones = jnp.ones((C,), jnp.float32)

    def fill_ones(i, carry):
        gbuf[0, i] = ones
        return carry

    lax.fori_loop(0, CH, fill_ones, 0)

    def fill_ones_p(i, carry):
        pvm[i] = ones
        return carry

    lax.fori_loop(0, RPT, fill_ones_p, 0)
    pltpu.sync_copy(pvm, agg.at[stripe])
    plsc.subcore_barrier()
    pend = [
        pltpu.async_copy(gbuf.at[0], agg.at[coli.at[j]], ssem, add=True)
        for j in range(NCHUNK)
    ]
    for d in pend:
        d.wait()
    plsc.subcore_barrier()
    exchange(0)

    pltpu.sync_copy(h_hbm.at[stripe], pvm)

    def prep_row(i, carry):
        d = avm[i] + xvm[i] - 1.0
        yi = lax.bitcast_convert_type(d, jnp.int32)
        yi = jnp.int32(0x5F3759DF) - (yi >> 1)
        y = lax.bitcast_convert_type(yi, jnp.float32)
        y = y * (1.5 - 0.5 * d * y * y)
        y = y * (1.5 - 0.5 * d * y * y)
        y = y * (1.5 - 0.5 * d * y * y)
        p0 = y * pvm[i]
        pvm[i] = p0
        wvm[i] = (1.0 - ALPHA) * y * y
        qvm[i] = ALPHA * p0
        avm[i] = d * y
        return carry

    lax.fori_loop(0, RPT, prep_row, 0)
    pltpu.sync_copy(avm, sinv_hbm.at[stripe])
    pltpu.sync_copy(pvm, pout_hbm.at[stripe])
    plsc.subcore_barrier()

    for r in range(K):
        pltpu.sync_copy(pout_hbm.at[stripe], pvm)
        pltpu.sync_copy(pvm, agg.at[stripe])
        plsc.subcore_barrier()

        def fire(j):
            return pltpu.async_copy(
                pout_hbm.at[rowi.at[j]], gbuf.at[j % 3], gsem)

        pend_g = {0: fire(0), 1: fire(1)}
        pend_s = {}
        for j in range(NCHUNK):
            if j + 2 < NCHUNK:
                if j - 1 in pend_s:
                    pend_s.pop(j - 1).wait()
                pend_g[j + 2] = fire(j + 2)
            pend_g.pop(j).wait()
            pend_s[j] = pltpu.async_copy(
                gbuf.at[j % 3], agg.at[coli.at[j]], ssem, add=True)
        for j in sorted(pend_s):
            pend_s[j].wait()
        plsc.subcore_barrier()

        exchange(r + 1)

        def dense_row(i, carry):
            pvm[i] = wvm[i] * (avm[i] + xvm[i] - pvm[i]) + qvm[i]
            return carry

        lax.fori_loop(0, RPT, dense_row, 0)
        pltpu.sync_copy(pvm, pout_hbm.at[stripe])
        plsc.subcore_barrier()


def kernel(x, edge_index, W1, b1, W2, b2):
    row = edge_index[0].reshape(NW, NCHUNK, CH)
    col = edge_index[1].reshape(NW, NCHUNK, CH)

    h = _mlp(x, W1.T, b1.reshape(1, HID), W2.T, b2.reshape(1, C))
    h = jnp.pad(h, ((0, NP - N), (0, 0)))

    p, sinv, _ = _sc_prop(row, col, h)

    return _final(p, sinv)[:N]
